# baseline (speedup 1.0000x reference)
import jax
import jax.numpy as jnp
from jax import lax
from jax.experimental import pallas as pl

H, W = 64, 64


def _raster_body(px_ref, py_ref, r0_ref, r1_ref, o0_ref, o1_ref, out_ref):
    px = px_ref[0]
    py = py_ref[0]
    rowf = py / r0_ref[0] + o0_ref[0]
    colf = px / r1_ref[0] + o1_ref[0]
    row = rowf.astype(jnp.int32)
    col = colf.astype(jnp.int32)
    valid = (row >= 0) & (row < H) & (col >= 0) & (col < W)
    P = px.shape[-1]
    hh = lax.broadcasted_iota(jnp.int32, (H, 1, P), 0)
    ww = lax.broadcasted_iota(jnp.int32, (1, W, P), 1)
    rowm = hh == row.reshape(1, 1, P)
    colm = (ww == col.reshape(1, 1, P)) & valid.reshape(1, 1, P)
    out_ref[0] = (rowm & colm).astype(jnp.float32)


def kernel(x, resolution, origin):
    B, S, n2 = x.shape
    P = n2 // 2
    N = B * S
    pts = x.reshape(N, P, 2)
    px = pts[:, :, 0].reshape(N, 1, P)
    py = pts[:, :, 1].reshape(N, 1, P)
    r0 = resolution[:, :, 0].reshape(N, 1, 1)
    r1 = resolution[:, :, 1].reshape(N, 1, 1)
    o0 = origin[:, :, 0].reshape(N, 1, 1)
    o1 = origin[:, :, 1].reshape(N, 1, 1)

    vec_spec = pl.BlockSpec((1, 1, P), lambda g: (g, 0, 0))
    scl_spec = pl.BlockSpec((1, 1, 1), lambda g: (g, 0, 0))
    out = pl.pallas_call(
        _raster_body,
        grid=(N,),
        in_specs=[vec_spec, vec_spec, scl_spec, scl_spec, scl_spec, scl_spec],
        out_specs=pl.BlockSpec((1, H, W, P), lambda g: (g, 0, 0, 0)),
        out_shape=jax.ShapeDtypeStruct((N, H, W, P), jnp.float32),
    )(px, py, r0, r1, o0, o1)
    return out.reshape(B, S, H, W, P)
